# baseline (speedup 1.0000x reference)
import functools

import jax
import jax.numpy as jnp
from jax import lax
from jax.experimental import pallas as pl
from jax.experimental.pallas import tpu as pltpu
from jax.experimental.pallas import tpu_sc as plsc

NBPS = 4
CLIP = 20.0
NC, NS = 2, 16
NW = NC * NS


def _build(B, K, S):
    bpw = B // NW
    mesh = plsc.VectorSubcoreMesh(core_axis_name="c", subcore_axis_name="s",
                                  num_cores=NC, num_subcores=NS)

    @functools.partial(
        pl.kernel,
        out_type=jax.ShapeDtypeStruct((B, S, NBPS), jnp.float32),
        mesh=mesh,
        scratch_types=[
            pltpu.VMEM((bpw, K, S), jnp.int32),
            pltpu.VMEM((bpw, K), jnp.float32),
            pltpu.VMEM((bpw, S, NBPS), jnp.float32),
            pltpu.VMEM((24,), jnp.float32),
        ],
        compiler_params=pltpu.CompilerParams(needs_layout_passes=False,
                                             use_tc_tiling_on_sc=False),
    )
    def llr_kernel(pi_hbm, d_hbm, out_hbm, pi_v, d_v, out_v, fold_v):
        wid = lax.axis_index("s") * NC + lax.axis_index("c")
        base = wid * bpw
        pltpu.sync_copy(pi_hbm.at[pl.ds(base, bpw)], pi_v)
        pltpu.sync_copy(d_hbm.at[pl.ds(base, bpw)], d_v)

        iota = lax.iota(jnp.int32, 16)
        hi = iota >> 3
        lane_s = iota & 7
        lane_lt8 = iota < 8
        inf = jnp.full((16,), jnp.inf, jnp.float32)
        splat_i = [jnp.full((16,), i, jnp.int32) for i in range(NBPS)]

        UNROLL = 4

        def row(b, carry):
            splat_b = jnp.zeros((16,), jnp.int32) + b

            def jstep(jc, accs):
                a0, a1 = list(accs[0]), list(accs[1])
                for u in range(UNROLL):
                    j = jc * UNROLL + u
                    ik = hi + 2 * j
                    v = plsc.load_gather(pi_v, [splat_b, ik, lane_s])
                    dj = plsc.load_gather(d_v, [splat_b, ik])
                    for i in range(NBPS):
                        m0 = (v & (8 >> i)) == 0
                        a0[i] = jnp.minimum(a0[i], jnp.where(m0, dj, inf))
                        a1[i] = jnp.minimum(a1[i], jnp.where(m0, inf, dj))
                return (tuple(a0), tuple(a1))

            a0, a1 = lax.fori_loop(0, K // 2 // UNROLL, jstep,
                                   ((inf,) * NBPS, (inf,) * NBPS))
            for i in range(NBPS):
                fold_v[pl.ds(0, 16)] = a0[i]
                f0 = jnp.minimum(a0[i], fold_v[pl.ds(8, 16)])
                fold_v[pl.ds(0, 16)] = a1[i]
                f1 = jnp.minimum(a1[i], fold_v[pl.ds(8, 16)])
                llr = jnp.clip((f0 - f1) * 0.5, -CLIP, CLIP)
                plsc.store_scatter(out_v, [splat_b, lane_s, splat_i[i]],
                                   llr, mask=lane_lt8)
            return carry

        lax.fori_loop(0, bpw, row, 0)
        pltpu.sync_copy(out_v, out_hbm.at[pl.ds(base, bpw)])

    return llr_kernel


def kernel(y, r, dists, path_inds, path_syms):
    B, K, S = path_inds.shape
    n_chunks = 4
    cb = B // n_chunks
    llr_fn = _build(cb, K, S)
    outs = []
    for c in range(n_chunks):
        pc = lax.slice_in_dim(path_inds, c * cb, (c + 1) * cb, axis=0)
        dc = lax.slice_in_dim(dists, c * cb, (c + 1) * cb, axis=0)
        outs.append(llr_fn(pc, dc))
    return jnp.concatenate(outs, axis=0)


# speedup vs baseline: 1.1562x; 1.1562x over previous
import functools

import jax
import jax.numpy as jnp
from jax import lax
from jax.experimental import pallas as pl
from jax.experimental.pallas import tpu as pltpu
from jax.experimental.pallas import tpu_sc as plsc

NBPS = 4
CLIP = 20.0
NC, NS = 2, 16
NW = NC * NS

def _pack_body(pi_ref, d_ref, pk_ref, dd_ref):
    x = pi_ref[...]
    s_iota = lax.broadcasted_iota(jnp.int32, x.shape, 2)
    pk_ref[...] = jnp.sum(x << (4 * s_iota), axis=-1)
    dd_ref[...] = d_ref[...]


def _build_pack(B, K, S, cb, bB, c):
    gpc = cb // bB
    return pl.pallas_call(
        _pack_body,
        grid=(gpc,),
        in_specs=[
            pl.BlockSpec((bB, K, S), lambda g, c=c, gpc=gpc: (c * gpc + g, 0, 0)),
            pl.BlockSpec((bB, K), lambda g, c=c, gpc=gpc: (c * gpc + g, 0)),
        ],
        out_specs=[
            pl.BlockSpec((bB, K), lambda g: (g, 0)),
            pl.BlockSpec((bB, K), lambda g: (g, 0)),
        ],
        out_shape=[
            jax.ShapeDtypeStruct((cb, K), jnp.int32),
            jax.ShapeDtypeStruct((cb, K), jnp.float32),
        ],
    )


def _build_sc(cb, K, S):
    bpw = cb // NW
    rows_out = bpw * S * NBPS // 128
    mesh = plsc.VectorSubcoreMesh(core_axis_name="c", subcore_axis_name="s",
                                  num_cores=NC, num_subcores=NS)

    @functools.partial(
        pl.kernel,
        out_type=jax.ShapeDtypeStruct((cb * S * NBPS // 128, 128), jnp.float32),
        mesh=mesh,
        scratch_types=[
            pltpu.VMEM((bpw, K), jnp.int32),
            pltpu.VMEM((bpw, K), jnp.float32),
            pltpu.VMEM((rows_out, 128), jnp.float32),
            pltpu.VMEM((24,), jnp.float32),
        ],
        compiler_params=pltpu.CompilerParams(needs_layout_passes=False,
                                             use_tc_tiling_on_sc=False),
    )
    def llr_kernel(pk_hbm, d_hbm, out_hbm, pk_v, d_v, out_v, fold_v):
        wid = lax.axis_index("s") * NC + lax.axis_index("c")
        pltpu.sync_copy(pk_hbm.at[pl.ds(wid * bpw, bpw)], pk_v)
        pltpu.sync_copy(d_hbm.at[pl.ds(wid * bpw, bpw)], d_v)

        iota = lax.iota(jnp.int32, 16)
        hi = iota >> 3
        lane_s = iota & 7
        sh4 = lane_s * 4
        lane_lt8 = iota < 8
        inf = jnp.full((16,), jnp.inf, jnp.float32)
        oidx = [(iota & 7) * NBPS + i for i in range(NBPS)]

        UNROLL = 4

        def row(b, carry):
            in_row = jnp.zeros((16,), jnp.int32) + b
            out_row = jnp.zeros((16,), jnp.int32) + (b >> 2)
            obase = (b & 3) * 32

            def jstep(jc, accs):
                a0, a1 = list(accs[0]), list(accs[1])
                for u in range(UNROLL):
                    j = jc * UNROLL + u
                    lane = hi + 2 * j
                    p = plsc.load_gather(pk_v, [in_row, lane])
                    dj = plsc.load_gather(d_v, [in_row, lane])
                    v = (p >> sh4) & 15
                    for i in range(NBPS):
                        m0 = (v & (8 >> i)) == 0
                        a0[i] = jnp.minimum(a0[i], jnp.where(m0, dj, inf))
                        a1[i] = jnp.minimum(a1[i], jnp.where(m0, inf, dj))
                return (tuple(a0), tuple(a1))

            a0, a1 = lax.fori_loop(0, K // 2 // UNROLL, jstep,
                                   ((inf,) * NBPS, (inf,) * NBPS))
            for i in range(NBPS):
                fold_v[pl.ds(0, 16)] = a0[i]
                f0 = jnp.minimum(a0[i], fold_v[pl.ds(8, 16)])
                fold_v[pl.ds(0, 16)] = a1[i]
                f1 = jnp.minimum(a1[i], fold_v[pl.ds(8, 16)])
                llr = jnp.clip((f0 - f1) * 0.5, -CLIP, CLIP)
                plsc.store_scatter(out_v, [out_row, oidx[i] + obase],
                                   llr, mask=lane_lt8)
            return carry

        lax.fori_loop(0, bpw, row, 0)
        pltpu.sync_copy(out_v, out_hbm.at[pl.ds(wid * rows_out, rows_out)])

    return llr_kernel


def kernel(y, r, dists, path_inds, path_syms):
    B, K, S = path_inds.shape
    n_chunks = 4
    cb = B // n_chunks
    bB = 256
    sc_fn = _build_sc(cb, K, S)
    outs = []
    for c in range(n_chunks):
        pk, dd = _build_pack(B, K, S, cb, bB, c)(path_inds, dists)
        outs.append(sc_fn(pk, dd))
    out = jnp.concatenate(outs, axis=0)
    return out.reshape(B, S, NBPS)


# speedup vs baseline: 1.4262x; 1.2335x over previous
import functools

import jax
import jax.numpy as jnp
from jax import lax
from jax.experimental import pallas as pl
from jax.experimental.pallas import tpu as pltpu
from jax.experimental.pallas import tpu_sc as plsc

NBPS = 4
CLIP = 20.0
NC, NS = 2, 16
NW = NC * NS


def _build_sc(B, K, S):
    bpw = B // NW
    pi_rows = bpw * K * S // 128
    d_rows = bpw * K // 128
    out_rows = bpw * S * NBPS // 128
    mesh = plsc.VectorSubcoreMesh(core_axis_name="c", subcore_axis_name="s",
                                  num_cores=NC, num_subcores=NS)

    @functools.partial(
        pl.kernel,
        out_type=jax.ShapeDtypeStruct((B * S * NBPS // 128, 128), jnp.float32),
        mesh=mesh,
        scratch_types=[
            pltpu.VMEM((pi_rows, 128), jnp.int32),
            pltpu.VMEM((d_rows, 128), jnp.float32),
            pltpu.VMEM((out_rows, 128), jnp.float32),
            pltpu.VMEM((24,), jnp.float32),
        ],
        compiler_params=pltpu.CompilerParams(needs_layout_passes=False,
                                             use_tc_tiling_on_sc=False),
    )
    def llr_kernel(pi_hbm, d_hbm, out_hbm, pi_v, d_v, out_v, fold_v):
        wid = lax.axis_index("s") * NC + lax.axis_index("c")
        pltpu.sync_copy(pi_hbm.at[pl.ds(wid * pi_rows, pi_rows)], pi_v)
        pltpu.sync_copy(d_hbm.at[pl.ds(wid * d_rows, d_rows)], d_v)

        iota = lax.iota(jnp.int32, 16)
        hi = iota >> 3
        lane_lt8 = iota < 8
        inf = jnp.full((16,), jnp.inf, jnp.float32)
        oidx = [(iota & 7) * NBPS + i for i in range(NBPS)]

        UNROLL = 8

        def row(b, carry):
            d_row = jnp.zeros((16,), jnp.int32) + (b >> 1)
            d_lane0 = (b & 1) * 64
            out_row = jnp.zeros((16,), jnp.int32) + (b >> 2)
            obase = (b & 3) * 32

            def jstep(jc, accs):
                a0, a1 = list(accs[0]), list(accs[1])
                pi_row = b * 4 + jc
                for u in range(UNROLL):
                    j = jc * UNROLL + u
                    v = pi_v[pi_row, pl.ds(16 * u, 16)]
                    dj = plsc.load_gather(d_v, [d_row, hi + (d_lane0 + 2 * j)])
                    for i in range(NBPS):
                        m0 = (v & (8 >> i)) == 0
                        a0[i] = jnp.minimum(a0[i], jnp.where(m0, dj, inf))
                        a1[i] = jnp.minimum(a1[i], jnp.where(m0, inf, dj))
                return (tuple(a0), tuple(a1))

            a0, a1 = lax.fori_loop(0, K // 2 // UNROLL, jstep,
                                   ((inf,) * NBPS, (inf,) * NBPS))
            for i in range(NBPS):
                fold_v[pl.ds(0, 16)] = a0[i]
                f0 = jnp.minimum(a0[i], fold_v[pl.ds(8, 16)])
                fold_v[pl.ds(0, 16)] = a1[i]
                f1 = jnp.minimum(a1[i], fold_v[pl.ds(8, 16)])
                llr = jnp.clip((f0 - f1) * 0.5, -CLIP, CLIP)
                plsc.store_scatter(out_v, [out_row, oidx[i] + obase],
                                   llr, mask=lane_lt8)
            return carry

        lax.fori_loop(0, bpw, row, 0)
        pltpu.sync_copy(out_v, out_hbm.at[pl.ds(wid * out_rows, out_rows)])

    return llr_kernel


def kernel(y, r, dists, path_inds, path_syms):
    B, K, S = path_inds.shape
    pi = path_inds.reshape(B * K * S // 128, 128)
    dd = dists.reshape(B * K // 128, 128)
    out = _build_sc(B, K, S)(pi, dd)
    return out.reshape(B, S, NBPS)
